# baseline (speedup 1.0000x reference)
import jax
import jax.numpy as jnp
from jax.experimental import pallas as pl
from jax.experimental.pallas import tpu as pltpu
from jax.experimental.pallas import tpu_sc as plsc

T_BLK = 512
S_BLK = 4096


def _argmin_kernel(tar_ref, src_ref, tsq_ref, ssq_ref, out_ref,
                   bv_ref, bc_ref, bv2_ref, bc2_ref):
    t = tar_ref[...]
    tsq = tsq_ref[...]
    n_src = src_ref.shape[1]

    def dist(c):
        s = src_ref[:, pl.ds(c * S_BLK, S_BLK)]
        mm2 = jax.lax.dot_general(
            t, s, (((1,), (0,)), ((), ())),
            preferred_element_type=jnp.float32)
        ssq = ssq_ref[:, pl.ds(c * S_BLK, S_BLK)]
        return (tsq + mm2) + ssq

    bv_ref[...] = dist(jnp.int32(0))
    bc_ref[...] = jnp.zeros((T_BLK, S_BLK), jnp.float32)
    bv2_ref[...] = dist(jnp.int32(1))
    bc2_ref[...] = jnp.full((T_BLK, S_BLK), 1.0, jnp.float32)

    def body(p, _):
        cA = 2 * p
        d2a = dist(cA)
        bva = bv_ref[...]
        maska = d2a < bva
        bv_ref[...] = jnp.where(maska, d2a, bva)
        bc_ref[...] = jnp.where(maska, cA.astype(jnp.float32), bc_ref[...])
        cB = 2 * p + 1
        d2b = dist(cB)
        bvb = bv2_ref[...]
        maskb = d2b < bvb
        bv2_ref[...] = jnp.where(maskb, d2b, bvb)
        bc2_ref[...] = jnp.where(maskb, cB.astype(jnp.float32), bc2_ref[...])
        return 0

    jax.lax.fori_loop(1, n_src // (2 * S_BLK), body, 0)

    bva, bvb = bv_ref[...], bv2_ref[...]
    bca, bcb = bc_ref[...], bc2_ref[...]
    takeb = (bvb < bva) | ((bvb == bva) & (bcb < bca))
    bv = jnp.where(takeb, bvb, bva)
    bc_ref[...] = jnp.where(takeb, bcb, bca)
    vmin = jnp.min(bv, axis=1, keepdims=True)
    lane = jax.lax.broadcasted_iota(jnp.int32, (T_BLK, S_BLK), 1)
    flat = bc_ref[...] * jnp.float32(S_BLK) + lane.astype(jnp.float32)
    cand = jnp.where(bv == vmin, flat, jnp.float32(1e9))
    idx = jnp.min(cand, axis=1)
    out_ref[...] = idx.astype(jnp.int32).reshape(T_BLK, 1)


def _nn_indices_pallas(src_V, tar_V):
    n_src = src_V.shape[0]
    n_tar = tar_V.shape[0]
    tsq = jnp.sum(tar_V * tar_V, axis=1).reshape(n_tar, 1)
    ssq = jnp.sum(src_V * src_V, axis=1).reshape(1, n_src)
    tar_bf = (-2.0 * tar_V).astype(jnp.bfloat16)
    src_bf = src_V.T.astype(jnp.bfloat16)
    idx = pl.pallas_call(
        _argmin_kernel,
        grid=(n_tar // T_BLK,),
        in_specs=[
            pl.BlockSpec((T_BLK, 3), lambda i: (i, 0)),
            pl.BlockSpec((3, n_src), lambda i: (0, 0)),
            pl.BlockSpec((T_BLK, 1), lambda i: (i, 0)),
            pl.BlockSpec((1, n_src), lambda i: (0, 0)),
        ],
        out_specs=pl.BlockSpec((T_BLK, 1), lambda i: (i, 0)),
        out_shape=jax.ShapeDtypeStruct((n_tar, 1), jnp.int32),
        scratch_shapes=[
            pltpu.VMEM((T_BLK, S_BLK), jnp.float32),
            pltpu.VMEM((T_BLK, S_BLK), jnp.float32),
            pltpu.VMEM((T_BLK, S_BLK), jnp.float32),
            pltpu.VMEM((T_BLK, S_BLK), jnp.float32),
        ],
    )(tar_bf, src_bf, tsq, ssq)
    return idx[:, 0]


_SC_UNITS = 32
_SC_LANES = 16


_SC_WIN = 128


def _sc_gather_loss(src_pad, tar_pad, idx2d):
    n_tar = tar_pad.shape[0]
    per = n_tar // _SC_UNITS
    n_win = per // _SC_WIN

    mesh = plsc.VectorSubcoreMesh(core_axis_name="c", subcore_axis_name="s")

    @pl.kernel(
        out_type=jax.ShapeDtypeStruct((_SC_UNITS, _SC_LANES), jnp.float32),
        mesh=mesh,
        scratch_types=[
            pltpu.VMEM((1, per), jnp.int32),
            pltpu.VMEM((_SC_WIN, 128), jnp.float32),
            pltpu.VMEM((per, _SC_LANES), jnp.float32),
            pltpu.VMEM((1, _SC_LANES), jnp.float32),
            pltpu.SemaphoreType.DMA,
            pltpu.SemaphoreType.DMA,
        ])
    def k(src_hbm, tar_hbm, idx_hbm, o_hbm, idxv, gv, tv, acc, sem1, sem2):
        ci = jax.lax.axis_index("c")
        si = jax.lax.axis_index("s")
        unit = ci * (_SC_UNITS // 2) + si
        base = unit * per
        cp_i = pltpu.async_copy(idx_hbm.at[:, pl.ds(base, per)], idxv, sem1)
        cp_t = pltpu.async_copy(tar_hbm.at[pl.ds(base, per), :], tv, sem2)
        cp_i.wait()
        cp_t.wait()
        acc[...] = jnp.zeros((1, _SC_LANES), jnp.float32)

        @pl.loop(0, n_win)
        def _(w):
            pltpu.sync_copy(src_hbm.at[idxv.at[0, pl.ds(w * _SC_WIN, _SC_WIN)]],
                            gv)
            @pl.loop(0, _SC_WIN)
            def _(r):
                d = (gv[pl.ds(r, 1), : _SC_LANES]
                     - tv[pl.ds(w * _SC_WIN + r, 1), :])
                acc[...] += d * d

        pltpu.sync_copy(acc, o_hbm.at[pl.ds(unit, 1), :])

    return k(src_pad, tar_pad, idx2d)


def kernel(src_V, tar_V):
    idx = _nn_indices_pallas(src_V, tar_V)
    src_pad = jnp.pad(src_V, ((0, 0), (0, 128 - src_V.shape[1])))
    tar_pad = jnp.pad(tar_V, ((0, 0), (0, _SC_LANES - tar_V.shape[1])))
    partials = _sc_gather_loss(src_pad, tar_pad, idx.reshape(1, -1))
    return 0.5 * jnp.sum(partials)


# speedup vs baseline: 1.1405x; 1.1405x over previous
import jax
import jax.numpy as jnp
from jax.experimental import pallas as pl
from jax.experimental.pallas import tpu as pltpu
from jax.experimental.pallas import tpu_sc as plsc

T_BLK = 1024
S_BLK = 4096


def _argmin_kernel(tar_ref, src_ref, tsq_ref, ssq_ref, out_ref,
                   bv_ref, bc_ref):
    t = tar_ref[...]
    tsq = tsq_ref[...]
    n_src = src_ref.shape[1]

    def dist(c):
        s = src_ref[:, pl.ds(c * S_BLK, S_BLK)]
        mm2 = jax.lax.dot_general(
            t, s, (((1,), (0,)), ((), ())),
            preferred_element_type=jnp.float32)
        ssq = ssq_ref[:, pl.ds(c * S_BLK, S_BLK)]
        return (tsq + mm2) + ssq

    bv_ref[...] = dist(jnp.int32(0))
    bc_ref[...] = jnp.zeros((T_BLK, S_BLK), jnp.float32)

    def body(c, _):
        d2 = dist(c)
        bv = bv_ref[...]
        mask = d2 < bv
        bv_ref[...] = jnp.where(mask, d2, bv)
        bc_ref[...] = jnp.where(mask, c.astype(jnp.float32), bc_ref[...])
        return 0

    jax.lax.fori_loop(1, n_src // S_BLK, body, 0)

    bv = bv_ref[...]
    vmin = jnp.min(bv, axis=1, keepdims=True)
    lane = jax.lax.broadcasted_iota(jnp.int32, (T_BLK, S_BLK), 1)
    flat = bc_ref[...] * jnp.float32(S_BLK) + lane.astype(jnp.float32)
    cand = jnp.where(bv == vmin, flat, jnp.float32(1e9))
    idx = jnp.min(cand, axis=1)
    out_ref[...] = idx.astype(jnp.int32).reshape(T_BLK, 1)


def _nn_indices_pallas(src_V, tar_V):
    n_src = src_V.shape[0]
    n_tar = tar_V.shape[0]
    tsq = jnp.sum(tar_V * tar_V, axis=1).reshape(n_tar, 1)
    ssq = jnp.sum(src_V * src_V, axis=1).reshape(1, n_src)
    tar_bf = (-2.0 * tar_V).astype(jnp.bfloat16)
    src_bf = src_V.T.astype(jnp.bfloat16)
    idx = pl.pallas_call(
        _argmin_kernel,
        grid=(n_tar // T_BLK,),
        in_specs=[
            pl.BlockSpec((T_BLK, 3), lambda i: (i, 0)),
            pl.BlockSpec((3, n_src), lambda i: (0, 0)),
            pl.BlockSpec((T_BLK, 1), lambda i: (i, 0)),
            pl.BlockSpec((1, n_src), lambda i: (0, 0)),
        ],
        out_specs=pl.BlockSpec((T_BLK, 1), lambda i: (i, 0)),
        out_shape=jax.ShapeDtypeStruct((n_tar, 1), jnp.int32),
        scratch_shapes=[
            pltpu.VMEM((T_BLK, S_BLK), jnp.float32),
            pltpu.VMEM((T_BLK, S_BLK), jnp.float32),
        ],
    )(tar_bf, src_bf, tsq, ssq)
    return idx[:, 0]


_SC_UNITS = 32
_SC_LANES = 16


_SC_WIN = 128


def _sc_gather_loss(src_pad, tar_pad, idx2d):
    n_tar = tar_pad.shape[0]
    per = n_tar // _SC_UNITS
    n_win = per // _SC_WIN

    mesh = plsc.VectorSubcoreMesh(core_axis_name="c", subcore_axis_name="s")

    @pl.kernel(
        out_type=jax.ShapeDtypeStruct((_SC_UNITS, _SC_LANES), jnp.float32),
        mesh=mesh,
        scratch_types=[
            pltpu.VMEM((1, per), jnp.int32),
            pltpu.VMEM((_SC_WIN, 128), jnp.float32),
            pltpu.VMEM((per, _SC_LANES), jnp.float32),
            pltpu.VMEM((1, _SC_LANES), jnp.float32),
            pltpu.SemaphoreType.DMA,
            pltpu.SemaphoreType.DMA,
        ])
    def k(src_hbm, tar_hbm, idx_hbm, o_hbm, idxv, gv, tv, acc, sem1, sem2):
        ci = jax.lax.axis_index("c")
        si = jax.lax.axis_index("s")
        unit = ci * (_SC_UNITS // 2) + si
        base = unit * per
        cp_i = pltpu.async_copy(idx_hbm.at[:, pl.ds(base, per)], idxv, sem1)
        cp_t = pltpu.async_copy(tar_hbm.at[pl.ds(base, per), :], tv, sem2)
        cp_i.wait()
        cp_t.wait()
        acc[...] = jnp.zeros((1, _SC_LANES), jnp.float32)

        @pl.loop(0, n_win)
        def _(w):
            pltpu.sync_copy(src_hbm.at[idxv.at[0, pl.ds(w * _SC_WIN, _SC_WIN)]],
                            gv)
            @pl.loop(0, _SC_WIN)
            def _(r):
                d = (gv[pl.ds(r, 1), : _SC_LANES]
                     - tv[pl.ds(w * _SC_WIN + r, 1), :])
                acc[...] += d * d

        pltpu.sync_copy(acc, o_hbm.at[pl.ds(unit, 1), :])

    return k(src_pad, tar_pad, idx2d)


def kernel(src_V, tar_V):
    idx = _nn_indices_pallas(src_V, tar_V)
    src_pad = jnp.pad(src_V, ((0, 0), (0, 128 - src_V.shape[1])))
    tar_pad = jnp.pad(tar_V, ((0, 0), (0, _SC_LANES - tar_V.shape[1])))
    partials = _sc_gather_loss(src_pad, tar_pad, idx.reshape(1, -1))
    return 0.5 * jnp.sum(partials)
